# Initial kernel scaffold; baseline (speedup 1.0000x reference)
import jax
import jax.numpy as jnp
from jax.experimental import pallas as pl


def kernel(x, edge_index, edge_attr, batch, atom_emb, bond_emb, eps, gin_w1, gin_b1, gin_bn1_g, gin_bn1_b, gin_w2, gin_b2, out_bn_g, out_bn_b, vn_w1, vn_b1, vn_bn1_g, vn_bn1_b, vn_w2, vn_b2, vn_bn2_g, vn_bn2_b):
    raise NotImplementedError("write your pallas kernel here")


# speedup vs baseline: 11.6851x; 11.6851x over previous
import functools

import jax
import jax.numpy as jnp
from jax import lax
from jax.experimental import pallas as pl
from jax.experimental.pallas import tpu as pltpu
from jax.experimental.pallas import tpu_sc as plsc

N = 100000
E = 3200000
B = 512
D = 9
H = 18
L = 3

BN = 512
NB = 196
NP = BN * NB
NC = 2
NS = 16
NW = NC * NS
EC = 128
E_CHUNKS = E // EC
E_ITERS = -(-E_CHUNKS // NW)
N_CHUNKS = NP // EC
N_ITERS = -(-N_CHUNKS // NW)
ROWS_PER_TILE = NP // NS
LUT_REP = 32


def _sc_msg_body(hl_h, src_h, dst_h, eidx_h, lut_h, batch_h, zeros_h,
                 agg_o, pool_o,
                 src_v, dst_v, eidx_v, bv, rows_v, msg_v, hv, e_v,
                 agg_s, pool_s, sem):
    c = lax.axis_index("c")
    s = lax.axis_index("s")
    wid = s * NC + c

    pltpu.sync_copy(zeros_h.at[pl.ds(s * ROWS_PER_TILE, ROWS_PER_TILE)],
                    agg_s.at[pl.ds(s * ROWS_PER_TILE, ROWS_PER_TILE)])

    @pl.when(s == 0)
    def _():
        pltpu.sync_copy(zeros_h.at[pl.ds(0, B)], pool_s)

    pltpu.sync_copy(zeros_h.at[pl.ds(0, EC)], msg_v)
    plsc.subcore_barrier()

    def edge_step(i, _):
        cid = wid + NW * i

        @pl.when(cid < E_CHUNKS)
        def _():
            base = cid * EC
            pltpu.sync_copy(src_h.at[pl.ds(base, EC)], src_v)
            pltpu.sync_copy(eidx_h.at[pl.ds(base, EC)], eidx_v)
            pltpu.sync_copy(dst_h.at[pl.ds(base, EC)], dst_v)
            cp1 = pltpu.async_copy(hl_h.at[src_v], rows_v, sem)
            cp2 = pltpu.async_copy(lut_h.at[eidx_v], e_v, sem)
            cp1.wait()
            cp2.wait()
            for i in range(EC):
                msg_v[i] = jnp.maximum(rows_v[i] + e_v[i], 0.0)
            pltpu.sync_copy(msg_v, agg_s.at[dst_v], add=True)
        return _

    lax.fori_loop(0, E_ITERS, edge_step, None)

    def node_step(i, _):
        cid = wid + NW * i

        @pl.when(cid < N_CHUNKS)
        def _():
            base = cid * EC
            pltpu.sync_copy(hl_h.at[pl.ds(base, EC)], hv)
            pltpu.sync_copy(batch_h.at[pl.ds(base, EC)], bv)
            pltpu.sync_copy(hv, pool_s.at[bv], add=True)
        return _

    lax.fori_loop(0, N_ITERS, node_step, None)

    plsc.subcore_barrier()
    pltpu.sync_copy(agg_s.at[pl.ds(s * ROWS_PER_TILE, ROWS_PER_TILE)],
                    agg_o.at[c, pl.ds(s * ROWS_PER_TILE, ROWS_PER_TILE)])

    @pl.when(s == 0)
    def _():
        pltpu.sync_copy(pool_s, pool_o.at[c])


def _message(hl, src, dst, eidx, lut16, batch_p, zeros):
    mesh = plsc.VectorSubcoreMesh(core_axis_name="c", subcore_axis_name="s")
    fn = pl.kernel(
        _sc_msg_body, mesh=mesh,
        compiler_params=pltpu.CompilerParams(use_tc_tiling_on_sc=False),
        out_type=[jax.ShapeDtypeStruct((NC, NP, 16), jnp.float32),
                  jax.ShapeDtypeStruct((NC, B, 16), jnp.float32)],
        scratch_types=[
            pltpu.VMEM((EC,), jnp.int32),
            pltpu.VMEM((EC,), jnp.int32),
            pltpu.VMEM((EC,), jnp.int32),
            pltpu.VMEM((EC,), jnp.int32),
            pltpu.VMEM((EC, 16), jnp.float32),
            pltpu.VMEM((EC, 16), jnp.float32),
            pltpu.VMEM((EC, 16), jnp.float32),
            pltpu.VMEM((EC, 16), jnp.float32),
            pltpu.VMEM_SHARED((NP, 16), jnp.float32),
            pltpu.VMEM_SHARED((B, 16), jnp.float32),
            pltpu.SemaphoreType.DMA,
        ],
    )
    return fn(hl, src, dst, eidx, lut16, batch_p, zeros)


def _embed_body(x_ref, tab_ref, o_ref):
    pid = pl.program_id(0)
    h = jnp.zeros((BN, 16), jnp.float32)
    for f in range(9):
        col = x_ref[:, f]
        oh = (col[:, None] ==
              lax.broadcasted_iota(jnp.int32, (BN, 120), 1)).astype(jnp.float32)
        h = h + jnp.dot(oh, tab_ref[f], preferred_element_type=jnp.float32)
    rid = pid * BN + lax.broadcasted_iota(jnp.int32, (BN, 16), 0)
    o_ref[...] = jnp.where(rid < N, h, 0.0)


def _embed(x_p, atom16):
    return pl.pallas_call(
        _embed_body,
        grid=(NB,),
        in_specs=[pl.BlockSpec((BN, 9), lambda i: (i, 0)),
                  pl.BlockSpec((9, 120, 16), lambda i: (0, 0, 0))],
        out_specs=pl.BlockSpec((BN, 16), lambda i: (i, 0)),
        out_shape=jax.ShapeDtypeStruct((NP, 16), jnp.float32),
    )(x_p, atom16)


def _phA_body(hl_ref, a0_ref, a1_ref, w1_ref, b1_ref, eps_ref,
              u_ref, s1_ref, s2_ref, s1a, s2a):
    pid = pl.program_id(0)
    t16 = (1.0 + eps_ref[0, 0]) * hl_ref[...] + a0_ref[0] + a1_ref[0]
    u_ref[...] = jnp.dot(t16, w1_ref[...],
                         preferred_element_type=jnp.float32) + b1_ref[...]
    ps1 = jnp.sum(t16, 0, keepdims=True)
    ps2 = lax.dot_general(t16, t16, (((0,), (0,)), ((), ())),
                          preferred_element_type=jnp.float32)

    @pl.when(pid == 0)
    def _():
        s1a[...] = jnp.zeros_like(s1a)
        s2a[...] = jnp.zeros_like(s2a)

    s1a[...] += ps1
    s2a[...] += ps2

    @pl.when(pid == NB - 1)
    def _():
        s1_ref[...] = s1a[...]
        s2_ref[...] = s2a[...]


def _phaseA(hl, agg, w1p, b1, eps_l):
    return pl.pallas_call(
        _phA_body,
        grid=(NB,),
        in_specs=[pl.BlockSpec((BN, 16), lambda i: (i, 0)),
                  pl.BlockSpec((1, BN, 16), lambda i: (0, i, 0)),
                  pl.BlockSpec((1, BN, 16), lambda i: (1, i, 0)),
                  pl.BlockSpec((16, H), lambda i: (0, 0)),
                  pl.BlockSpec((1, H), lambda i: (0, 0)),
                  pl.BlockSpec((1, 1), lambda i: (0, 0))],
        out_specs=[pl.BlockSpec((BN, H), lambda i: (i, 0)),
                   pl.BlockSpec((1, 16), lambda i: (0, 0)),
                   pl.BlockSpec((16, 16), lambda i: (0, 0))],
        out_shape=[jax.ShapeDtypeStruct((NP, H), jnp.float32),
                   jax.ShapeDtypeStruct((1, 16), jnp.float32),
                   jax.ShapeDtypeStruct((16, 16), jnp.float32)],
        scratch_shapes=[pltpu.VMEM((1, 16), jnp.float32),
                        pltpu.VMEM((16, 16), jnp.float32)],
    )(hl, agg, agg, w1p, b1, eps_l)


def _phB_body(u_ref, s1_ref, s2_ref, w1_ref, b1_ref, g1_ref, bb1_ref,
              w2_ref, b2_ref, v_ref, z1_ref, z2_ref, z1a, z2a):
    pid = pl.program_id(0)
    m16 = s1_ref[...] / N
    w1 = w1_ref[...]
    mu = jnp.dot(m16, w1, preferred_element_type=jnp.float32) + b1_ref[...]
    C = s2_ref[...] / N - lax.dot_general(m16, m16, (((0,), (0,)), ((), ())),
                                          preferred_element_type=jnp.float32)
    varu = jnp.sum(w1 * jnp.dot(C, w1, preferred_element_type=jnp.float32),
                   0, keepdims=True)
    z = jnp.maximum(g1_ref[...] * (u_ref[...] - mu) * lax.rsqrt(varu + 1e-5)
                    + bb1_ref[...], 0.0)
    rid = pid * BN + lax.broadcasted_iota(jnp.int32, (BN, H), 0)
    z = jnp.where(rid < N, z, 0.0)
    v_ref[...] = jnp.dot(z, w2_ref[...],
                         preferred_element_type=jnp.float32) + b2_ref[...]
    pz1 = jnp.sum(z, 0, keepdims=True)
    pz2 = lax.dot_general(z, z, (((0,), (0,)), ((), ())),
                          preferred_element_type=jnp.float32)

    @pl.when(pid == 0)
    def _():
        z1a[...] = jnp.zeros_like(z1a)
        z2a[...] = jnp.zeros_like(z2a)

    z1a[...] += pz1
    z2a[...] += pz2

    @pl.when(pid == NB - 1)
    def _():
        z1_ref[...] = z1a[...]
        z2_ref[...] = z2a[...]


def _phaseB(u, s1, s2, w1p, b1, g1, bb1, w2p, b2p):
    return pl.pallas_call(
        _phB_body,
        grid=(NB,),
        in_specs=[pl.BlockSpec((BN, H), lambda i: (i, 0)),
                  pl.BlockSpec((1, 16), lambda i: (0, 0)),
                  pl.BlockSpec((16, 16), lambda i: (0, 0)),
                  pl.BlockSpec((16, H), lambda i: (0, 0)),
                  pl.BlockSpec((1, H), lambda i: (0, 0)),
                  pl.BlockSpec((1, H), lambda i: (0, 0)),
                  pl.BlockSpec((1, H), lambda i: (0, 0)),
                  pl.BlockSpec((H, 16), lambda i: (0, 0)),
                  pl.BlockSpec((1, 16), lambda i: (0, 0))],
        out_specs=[pl.BlockSpec((BN, 16), lambda i: (i, 0)),
                   pl.BlockSpec((1, H), lambda i: (0, 0)),
                   pl.BlockSpec((H, H), lambda i: (0, 0))],
        out_shape=[jax.ShapeDtypeStruct((NP, 16), jnp.float32),
                   jax.ShapeDtypeStruct((1, H), jnp.float32),
                   jax.ShapeDtypeStruct((H, H), jnp.float32)],
        scratch_shapes=[pltpu.VMEM((1, H), jnp.float32),
                        pltpu.VMEM((H, H), jnp.float32)],
    )(u, s1, s2, w1p, b1, g1, bb1, w2p, b2p)


def _phC_mid_body(v_ref, hl_ref, z1_ref, z2_ref, w2_ref, b2_ref, g2_ref,
                  bb2_ref, vn_ref, batch_ref, y_ref):
    pid = pl.program_id(0)
    mz = z1_ref[...] / N
    w2 = w2_ref[...]
    mv = jnp.dot(mz, w2, preferred_element_type=jnp.float32) + b2_ref[...]
    Cz = z2_ref[...] / N - lax.dot_general(mz, mz, (((0,), (0,)), ((), ())),
                                           preferred_element_type=jnp.float32)
    varv = jnp.sum(w2 * jnp.dot(Cz, w2, preferred_element_type=jnp.float32),
                   0, keepdims=True)
    y = g2_ref[...] * (v_ref[...] - mv) * lax.rsqrt(varv + 1e-5) + bb2_ref[...]
    y = jnp.maximum(y, 0.0)
    y = y + hl_ref[...]
    oh = (batch_ref[...][:, None] ==
          lax.broadcasted_iota(jnp.int32, (BN, B), 1)).astype(jnp.float32)
    y = y + jnp.dot(oh, vn_ref[...], preferred_element_type=jnp.float32)
    rid = pid * BN + lax.broadcasted_iota(jnp.int32, (BN, 16), 0)
    y_ref[...] = jnp.where(rid < N, y, 0.0)


def _phaseC_mid(v, hl, z1, z2, w2p, b2p, g2p, bb2p, vn_new, batch_p):
    return pl.pallas_call(
        _phC_mid_body,
        grid=(NB,),
        in_specs=[pl.BlockSpec((BN, 16), lambda i: (i, 0)),
                  pl.BlockSpec((BN, 16), lambda i: (i, 0)),
                  pl.BlockSpec((1, H), lambda i: (0, 0)),
                  pl.BlockSpec((H, H), lambda i: (0, 0)),
                  pl.BlockSpec((H, 16), lambda i: (0, 0)),
                  pl.BlockSpec((1, 16), lambda i: (0, 0)),
                  pl.BlockSpec((1, 16), lambda i: (0, 0)),
                  pl.BlockSpec((1, 16), lambda i: (0, 0)),
                  pl.BlockSpec((B, 16), lambda i: (0, 0)),
                  pl.BlockSpec((BN,), lambda i: (i,))],
        out_specs=pl.BlockSpec((BN, 16), lambda i: (i, 0)),
        out_shape=jax.ShapeDtypeStruct((NP, 16), jnp.float32),
    )(v, hl, z1, z2, w2p, b2p, g2p, bb2p, vn_new, batch_p)


def _phC_last_body(v_ref, hl_ref, z1_ref, z2_ref, w2_ref, b2_ref, g2_ref,
                   bb2_ref, o_ref, acc):
    pid = pl.program_id(0)
    mz = z1_ref[...] / N
    w2 = w2_ref[...]
    mv = jnp.dot(mz, w2, preferred_element_type=jnp.float32) + b2_ref[...]
    Cz = z2_ref[...] / N - lax.dot_general(mz, mz, (((0,), (0,)), ((), ())),
                                           preferred_element_type=jnp.float32)
    varv = jnp.sum(w2 * jnp.dot(Cz, w2, preferred_element_type=jnp.float32),
                   0, keepdims=True)
    y = g2_ref[...] * (v_ref[...] - mv) * lax.rsqrt(varv + 1e-5) + bb2_ref[...]
    y = y + hl_ref[...]
    rid = pid * BN + lax.broadcasted_iota(jnp.int32, (BN, 16), 0)
    y = jnp.where(rid < N, y, 0.0)

    @pl.when(pid == 0)
    def _():
        acc[...] = jnp.zeros_like(acc)

    acc[...] += jnp.sum(y, 0, keepdims=True)

    @pl.when(pid == NB - 1)
    def _():
        o_ref[...] = acc[...]


def _phaseC_last(v, hl, z1, z2, w2p, b2p, g2p, bb2p):
    return pl.pallas_call(
        _phC_last_body,
        grid=(NB,),
        in_specs=[pl.BlockSpec((BN, 16), lambda i: (i, 0)),
                  pl.BlockSpec((BN, 16), lambda i: (i, 0)),
                  pl.BlockSpec((1, H), lambda i: (0, 0)),
                  pl.BlockSpec((H, H), lambda i: (0, 0)),
                  pl.BlockSpec((H, 16), lambda i: (0, 0)),
                  pl.BlockSpec((1, 16), lambda i: (0, 0)),
                  pl.BlockSpec((1, 16), lambda i: (0, 0)),
                  pl.BlockSpec((1, 16), lambda i: (0, 0))],
        out_specs=pl.BlockSpec((1, 16), lambda i: (0, 0)),
        out_shape=jax.ShapeDtypeStruct((1, 16), jnp.float32),
        scratch_shapes=[pltpu.VMEM((1, 16), jnp.float32)],
    )(v, hl, z1, z2, w2p, b2p, g2p, bb2p)


def _vn_body(pool_ref, vn_ref, w1_ref, b1_ref, g1_ref, bb1_ref,
             w2_ref, b2_ref, g2_ref, bb2_ref, o_ref):
    vt = pool_ref[0] + pool_ref[1] + vn_ref[...]
    uu = jnp.dot(vt, w1_ref[...], preferred_element_type=jnp.float32) + b1_ref[...]
    mu = jnp.mean(uu, 0, keepdims=True)
    var = jnp.mean(uu * uu, 0, keepdims=True) - mu * mu
    uu = jnp.maximum(g1_ref[...] * (uu - mu) * lax.rsqrt(var + 1e-5)
                     + bb1_ref[...], 0.0)
    u2 = jnp.dot(uu, w2_ref[...], preferred_element_type=jnp.float32) + b2_ref[...]
    mu2 = jnp.mean(u2, 0, keepdims=True)
    var2 = jnp.mean(u2 * u2, 0, keepdims=True) - mu2 * mu2
    u2 = jnp.maximum(g2_ref[...] * (u2 - mu2) * lax.rsqrt(var2 + 1e-5)
                     + bb2_ref[...], 0.0)
    o_ref[...] = vn_ref[...] + u2


def _vn_update(pool, vn, vw1p, vb1, vg1, vbb1, vw2p, vb2p, vg2p, vbb2p):
    return pl.pallas_call(
        _vn_body,
        out_shape=jax.ShapeDtypeStruct((B, 16), jnp.float32),
    )(pool, vn, vw1p, vb1, vg1, vbb1, vw2p, vb2p, vg2p, vbb2p)


def kernel(x, edge_index, edge_attr, batch, atom_emb, bond_emb, eps, gin_w1,
           gin_b1, gin_bn1_g, gin_bn1_b, gin_w2, gin_b2, out_bn_g, out_bn_b,
           vn_w1, vn_b1, vn_bn1_g, vn_bn1_b, vn_w2, vn_b2, vn_bn2_g, vn_bn2_b):
    src = edge_index[0]
    dst = edge_index[1]

    lut16 = jnp.tile(jnp.pad(
        (bond_emb[0][:5][:, None, None, :] + bond_emb[1][:5][None, :, None, :]
         + bond_emb[2][:5][None, None, :, :]).reshape(125, D),
        ((0, 0), (0, 7))), (LUT_REP, 1))
    eidx = (edge_attr[:, 0] * 25 + edge_attr[:, 1] * 5 + edge_attr[:, 2]
            + 125 * (jnp.arange(E, dtype=jnp.int32) % LUT_REP))
    x_p = jnp.pad(x, ((0, NP - N), (0, 0)))
    batch_p = jnp.pad(batch, (0, NP - N))
    zeros = jnp.zeros((NP, 16), jnp.float32)
    atom16 = jnp.pad(atom_emb, ((0, 0), (0, 0), (0, 7)))
    w1p = jnp.pad(gin_w1, ((0, 0), (0, 7), (0, 0)))
    b1 = gin_b1[:, None, :]
    g1 = gin_bn1_g[:, None, :]
    bb1 = gin_bn1_b[:, None, :]
    w2p = jnp.pad(gin_w2, ((0, 0), (0, 0), (0, 7)))
    b2p = jnp.pad(gin_b2, ((0, 0), (0, 7)))[:, None, :]
    g2p = jnp.pad(out_bn_g, ((0, 0), (0, 7)))[:, None, :]
    bb2p = jnp.pad(out_bn_b, ((0, 0), (0, 7)))[:, None, :]
    vw1p = jnp.pad(vn_w1, ((0, 0), (0, 7), (0, 0)))
    vb1 = vn_b1[:, None, :]
    vg1 = vn_bn1_g[:, None, :]
    vbb1 = vn_bn1_b[:, None, :]
    vw2p = jnp.pad(vn_w2, ((0, 0), (0, 0), (0, 7)))
    vb2p = jnp.pad(vn_b2, ((0, 0), (0, 7)))[:, None, :]
    vg2p = jnp.pad(vn_bn2_g, ((0, 0), (0, 7)))[:, None, :]
    vbb2p = jnp.pad(vn_bn2_b, ((0, 0), (0, 7)))[:, None, :]

    hl = _embed(x_p, atom16)
    vn = jnp.zeros((B, 16), jnp.float32)
    out = None
    for l in range(L):
        agg, pool = _message(hl, src, dst, eidx, lut16, batch_p, zeros)
        eps_l = eps[l].reshape(1, 1)
        u, s1, s2 = _phaseA(hl, agg, w1p[l], b1[l], eps_l)
        v, z1, z2 = _phaseB(u, s1, s2, w1p[l], b1[l], g1[l], bb1[l],
                            w2p[l], b2p[l])
        if l < L - 1:
            vn = _vn_update(pool, vn, vw1p[l], vb1[l], vg1[l], vbb1[l],
                            vw2p[l], vb2p[l], vg2p[l], vbb2p[l])
            hl = _phaseC_mid(v, hl, z1, z2, w2p[l], b2p[l], g2p[l], bb2p[l],
                             vn, batch_p)
        else:
            out = _phaseC_last(v, hl, z1, z2, w2p[l], b2p[l], g2p[l], bb2p[l])
    return out[:, :D]


# speedup vs baseline: 19.4873x; 1.6677x over previous
import functools

import jax
import jax.numpy as jnp
from jax import lax
from jax.experimental import pallas as pl
from jax.experimental.pallas import tpu as pltpu
from jax.experimental.pallas import tpu_sc as plsc

N = 100000
E = 3200000
B = 512
D = 9
H = 18
L = 3

BN = 512
NB = 196
NP = BN * NB
NC = 2
NS = 16
NW = NC * NS
EC = 128
SK = 2
SE = SK * EC
SUP = E // SE
SBODY = -(-(-(-SUP // NW)) // 2)
PEND0 = SUP % NW
assert SUP % (2 * NW) == PEND0 and 0 < PEND0 <= NW
N_SUP = NP // SE
N_ITERS = -(-N_SUP // NW)
ROWS_PER_TILE = NP // NS
LUT_REP = 32


def _sc_msg_body(hl_h, sei_h, dst_h, lut_h, batch_h, zeros_h,
                 agg_o, pool_o,
                 idx_v, dst_v, rows_v, msg_v,
                 agg_s, pool_s, sem_g0, sem_g1, sem_s0, sem_s1):
    c = lax.axis_index("c")
    s = lax.axis_index("s")
    wid = s * NC + c
    sem_g = (sem_g0, sem_g1)
    sem_s = (sem_s0, sem_s1)

    pltpu.sync_copy(zeros_h.at[pl.ds(s * ROWS_PER_TILE, ROWS_PER_TILE)],
                    agg_s.at[pl.ds(s * ROWS_PER_TILE, ROWS_PER_TILE)])

    @pl.when(s == 0)
    def _():
        pltpu.sync_copy(zeros_h.at[pl.ds(0, B)], pool_s)

    plsc.subcore_barrier()

    def issue_gathers(p, cid):
        pltpu.sync_copy(sei_h.at[cid], idx_v.at[p])
        pltpu.sync_copy(dst_h.at[cid], dst_v.at[p])
        for j in range(SK):
            pltpu.async_copy(hl_h.at[idx_v.at[p, pl.ds(j * EC, EC)]],
                             rows_v.at[p, pl.ds(j * EC, EC)], sem_g[p])
            pltpu.async_copy(lut_h.at[idx_v.at[p, pl.ds(SE + j * EC, EC)]],
                             msg_v.at[p, pl.ds(j * EC, EC)], sem_g[p])

    def drain_gathers(p):
        pltpu.make_async_copy(zeros_h.at[pl.ds(0, SE)], rows_v.at[p],
                              sem_g[p]).wait()
        pltpu.make_async_copy(zeros_h.at[pl.ds(0, SE)], msg_v.at[p],
                              sem_g[p]).wait()

    def issue_scatters(p):
        for j in range(SK):
            pltpu.async_copy(msg_v.at[p, pl.ds(j * EC, EC)],
                             agg_s.at[dst_v.at[p, j]], sem_s[p], add=True)

    def drain_scatters(p):
        pltpu.make_async_copy(zeros_h.at[pl.ds(0, SE)], msg_v.at[p],
                              sem_s[p]).wait()

    def compute(p):
        def crow(j, carry):
            r0 = j * 16
            for t in range(16):
                r = r0 + t
                msg_v[p, r] = jnp.maximum(rows_v[p, r] + msg_v[p, r], 0.0)
            return carry
        lax.fori_loop(0, SE // 16, crow, None)

    issue_gathers(0, wid)

    def body(k, carry):
        cid_a = wid + NW * 2 * k
        cid_b = cid_a + NW
        cid_n = cid_a + 2 * NW

        @pl.when(jnp.logical_and(k > 0, cid_a - NW < SUP))
        def _():
            drain_scatters(1)

        @pl.when(cid_a < SUP)
        def _():
            drain_gathers(0)

            @pl.when(cid_b < SUP)
            def _():
                issue_gathers(1, cid_b)

            compute(0)
            issue_scatters(0)

        @pl.when(cid_b < SUP)
        def _():
            drain_gathers(1)
            drain_scatters(0)

            @pl.when(cid_n < SUP)
            def _():
                issue_gathers(0, cid_n)

            compute(1)
            issue_scatters(1)
        return carry

    lax.fori_loop(0, SBODY, body, None)

    @pl.when(wid < PEND0)
    def _():
        drain_scatters(0)

    def node_step(i, carry):
        cid = wid + NW * i

        @pl.when(cid < N_SUP)
        def _():
            pltpu.sync_copy(hl_h.at[pl.ds(cid * SE, SE)], rows_v.at[0])
            pltpu.sync_copy(batch_h.at[cid], dst_v.at[0])
            for j in range(SK):
                pltpu.sync_copy(rows_v.at[0, pl.ds(j * EC, EC)],
                                pool_s.at[dst_v.at[0, j]], add=True)
        return carry

    lax.fori_loop(0, N_ITERS, node_step, None)

    plsc.subcore_barrier()
    pltpu.sync_copy(agg_s.at[pl.ds(s * ROWS_PER_TILE, ROWS_PER_TILE)],
                    agg_o.at[c, pl.ds(s * ROWS_PER_TILE, ROWS_PER_TILE)])

    @pl.when(s == 0)
    def _():
        pltpu.sync_copy(pool_s, pool_o.at[c])


def _message(hl, sei, dst3, lut16, batch3, zeros):
    mesh = plsc.VectorSubcoreMesh(core_axis_name="c", subcore_axis_name="s")
    fn = pl.kernel(
        _sc_msg_body, mesh=mesh,
        compiler_params=pltpu.CompilerParams(use_tc_tiling_on_sc=False),
        out_type=[jax.ShapeDtypeStruct((NC, NP, 16), jnp.float32),
                  jax.ShapeDtypeStruct((NC, B, 16), jnp.float32)],
        scratch_types=[
            pltpu.VMEM((2, 2 * SE), jnp.int32),
            pltpu.VMEM((2, SK, EC), jnp.int32),
            pltpu.VMEM((2, SE, 16), jnp.float32),
            pltpu.VMEM((2, SE, 16), jnp.float32),
            pltpu.VMEM_SHARED((NP, 16), jnp.float32),
            pltpu.VMEM_SHARED((B, 16), jnp.float32),
            pltpu.SemaphoreType.DMA,
            pltpu.SemaphoreType.DMA,
            pltpu.SemaphoreType.DMA,
            pltpu.SemaphoreType.DMA,
        ],
    )
    return fn(hl, sei, dst3, lut16, batch3, zeros)


def _embed_body(x_ref, tab_ref, o_ref):
    pid = pl.program_id(0)
    h = jnp.zeros((BN, 16), jnp.float32)
    for f in range(9):
        col = x_ref[:, f]
        oh = (col[:, None] ==
              lax.broadcasted_iota(jnp.int32, (BN, 120), 1)).astype(jnp.float32)
        h = h + jnp.dot(oh, tab_ref[f], preferred_element_type=jnp.float32)
    rid = pid * BN + lax.broadcasted_iota(jnp.int32, (BN, 16), 0)
    o_ref[...] = jnp.where(rid < N, h, 0.0)


def _embed(x_p, atom16):
    return pl.pallas_call(
        _embed_body,
        grid=(NB,),
        in_specs=[pl.BlockSpec((BN, 9), lambda i: (i, 0)),
                  pl.BlockSpec((9, 120, 16), lambda i: (0, 0, 0))],
        out_specs=pl.BlockSpec((BN, 16), lambda i: (i, 0)),
        out_shape=jax.ShapeDtypeStruct((NP, 16), jnp.float32),
    )(x_p, atom16)


def _phA_body(hl_ref, a0_ref, a1_ref, w1_ref, b1_ref, eps_ref,
              u_ref, s1_ref, s2_ref, s1a, s2a):
    pid = pl.program_id(0)
    t16 = (1.0 + eps_ref[0, 0]) * hl_ref[...] + a0_ref[0] + a1_ref[0]
    u_ref[...] = jnp.dot(t16, w1_ref[...],
                         preferred_element_type=jnp.float32) + b1_ref[...]
    ps1 = jnp.sum(t16, 0, keepdims=True)
    ps2 = lax.dot_general(t16, t16, (((0,), (0,)), ((), ())),
                          preferred_element_type=jnp.float32)

    @pl.when(pid == 0)
    def _():
        s1a[...] = jnp.zeros_like(s1a)
        s2a[...] = jnp.zeros_like(s2a)

    s1a[...] += ps1
    s2a[...] += ps2

    @pl.when(pid == NB - 1)
    def _():
        s1_ref[...] = s1a[...]
        s2_ref[...] = s2a[...]


def _phaseA(hl, agg, w1p, b1, eps_l):
    return pl.pallas_call(
        _phA_body,
        grid=(NB,),
        in_specs=[pl.BlockSpec((BN, 16), lambda i: (i, 0)),
                  pl.BlockSpec((1, BN, 16), lambda i: (0, i, 0)),
                  pl.BlockSpec((1, BN, 16), lambda i: (1, i, 0)),
                  pl.BlockSpec((16, H), lambda i: (0, 0)),
                  pl.BlockSpec((1, H), lambda i: (0, 0)),
                  pl.BlockSpec((1, 1), lambda i: (0, 0))],
        out_specs=[pl.BlockSpec((BN, H), lambda i: (i, 0)),
                   pl.BlockSpec((1, 16), lambda i: (0, 0)),
                   pl.BlockSpec((16, 16), lambda i: (0, 0))],
        out_shape=[jax.ShapeDtypeStruct((NP, H), jnp.float32),
                   jax.ShapeDtypeStruct((1, 16), jnp.float32),
                   jax.ShapeDtypeStruct((16, 16), jnp.float32)],
        scratch_shapes=[pltpu.VMEM((1, 16), jnp.float32),
                        pltpu.VMEM((16, 16), jnp.float32)],
    )(hl, agg, agg, w1p, b1, eps_l)


def _phB_body(u_ref, s1_ref, s2_ref, w1_ref, b1_ref, g1_ref, bb1_ref,
              w2_ref, b2_ref, v_ref, z1_ref, z2_ref, z1a, z2a):
    pid = pl.program_id(0)
    m16 = s1_ref[...] / N
    w1 = w1_ref[...]
    mu = jnp.dot(m16, w1, preferred_element_type=jnp.float32) + b1_ref[...]
    C = s2_ref[...] / N - lax.dot_general(m16, m16, (((0,), (0,)), ((), ())),
                                          preferred_element_type=jnp.float32)
    varu = jnp.sum(w1 * jnp.dot(C, w1, preferred_element_type=jnp.float32),
                   0, keepdims=True)
    z = jnp.maximum(g1_ref[...] * (u_ref[...] - mu) * lax.rsqrt(varu + 1e-5)
                    + bb1_ref[...], 0.0)
    rid = pid * BN + lax.broadcasted_iota(jnp.int32, (BN, H), 0)
    z = jnp.where(rid < N, z, 0.0)
    v_ref[...] = jnp.dot(z, w2_ref[...],
                         preferred_element_type=jnp.float32) + b2_ref[...]
    pz1 = jnp.sum(z, 0, keepdims=True)
    pz2 = lax.dot_general(z, z, (((0,), (0,)), ((), ())),
                          preferred_element_type=jnp.float32)

    @pl.when(pid == 0)
    def _():
        z1a[...] = jnp.zeros_like(z1a)
        z2a[...] = jnp.zeros_like(z2a)

    z1a[...] += pz1
    z2a[...] += pz2

    @pl.when(pid == NB - 1)
    def _():
        z1_ref[...] = z1a[...]
        z2_ref[...] = z2a[...]


def _phaseB(u, s1, s2, w1p, b1, g1, bb1, w2p, b2p):
    return pl.pallas_call(
        _phB_body,
        grid=(NB,),
        in_specs=[pl.BlockSpec((BN, H), lambda i: (i, 0)),
                  pl.BlockSpec((1, 16), lambda i: (0, 0)),
                  pl.BlockSpec((16, 16), lambda i: (0, 0)),
                  pl.BlockSpec((16, H), lambda i: (0, 0)),
                  pl.BlockSpec((1, H), lambda i: (0, 0)),
                  pl.BlockSpec((1, H), lambda i: (0, 0)),
                  pl.BlockSpec((1, H), lambda i: (0, 0)),
                  pl.BlockSpec((H, 16), lambda i: (0, 0)),
                  pl.BlockSpec((1, 16), lambda i: (0, 0))],
        out_specs=[pl.BlockSpec((BN, 16), lambda i: (i, 0)),
                   pl.BlockSpec((1, H), lambda i: (0, 0)),
                   pl.BlockSpec((H, H), lambda i: (0, 0))],
        out_shape=[jax.ShapeDtypeStruct((NP, 16), jnp.float32),
                   jax.ShapeDtypeStruct((1, H), jnp.float32),
                   jax.ShapeDtypeStruct((H, H), jnp.float32)],
        scratch_shapes=[pltpu.VMEM((1, H), jnp.float32),
                        pltpu.VMEM((H, H), jnp.float32)],
    )(u, s1, s2, w1p, b1, g1, bb1, w2p, b2p)


def _phC_mid_body(v_ref, hl_ref, z1_ref, z2_ref, w2_ref, b2_ref, g2_ref,
                  bb2_ref, vn_ref, batch_ref, y_ref):
    pid = pl.program_id(0)
    mz = z1_ref[...] / N
    w2 = w2_ref[...]
    mv = jnp.dot(mz, w2, preferred_element_type=jnp.float32) + b2_ref[...]
    Cz = z2_ref[...] / N - lax.dot_general(mz, mz, (((0,), (0,)), ((), ())),
                                           preferred_element_type=jnp.float32)
    varv = jnp.sum(w2 * jnp.dot(Cz, w2, preferred_element_type=jnp.float32),
                   0, keepdims=True)
    y = g2_ref[...] * (v_ref[...] - mv) * lax.rsqrt(varv + 1e-5) + bb2_ref[...]
    y = jnp.maximum(y, 0.0)
    y = y + hl_ref[...]
    oh = (batch_ref[...][:, None] ==
          lax.broadcasted_iota(jnp.int32, (BN, B), 1)).astype(jnp.float32)
    y = y + jnp.dot(oh, vn_ref[...], preferred_element_type=jnp.float32)
    rid = pid * BN + lax.broadcasted_iota(jnp.int32, (BN, 16), 0)
    y_ref[...] = jnp.where(rid < N, y, 0.0)


def _phaseC_mid(v, hl, z1, z2, w2p, b2p, g2p, bb2p, vn_new, batch_p):
    return pl.pallas_call(
        _phC_mid_body,
        grid=(NB,),
        in_specs=[pl.BlockSpec((BN, 16), lambda i: (i, 0)),
                  pl.BlockSpec((BN, 16), lambda i: (i, 0)),
                  pl.BlockSpec((1, H), lambda i: (0, 0)),
                  pl.BlockSpec((H, H), lambda i: (0, 0)),
                  pl.BlockSpec((H, 16), lambda i: (0, 0)),
                  pl.BlockSpec((1, 16), lambda i: (0, 0)),
                  pl.BlockSpec((1, 16), lambda i: (0, 0)),
                  pl.BlockSpec((1, 16), lambda i: (0, 0)),
                  pl.BlockSpec((B, 16), lambda i: (0, 0)),
                  pl.BlockSpec((BN,), lambda i: (i,))],
        out_specs=pl.BlockSpec((BN, 16), lambda i: (i, 0)),
        out_shape=jax.ShapeDtypeStruct((NP, 16), jnp.float32),
    )(v, hl, z1, z2, w2p, b2p, g2p, bb2p, vn_new, batch_p)


def _phC_last_body(v_ref, hl_ref, z1_ref, z2_ref, w2_ref, b2_ref, g2_ref,
                   bb2_ref, o_ref, acc):
    pid = pl.program_id(0)
    mz = z1_ref[...] / N
    w2 = w2_ref[...]
    mv = jnp.dot(mz, w2, preferred_element_type=jnp.float32) + b2_ref[...]
    Cz = z2_ref[...] / N - lax.dot_general(mz, mz, (((0,), (0,)), ((), ())),
                                           preferred_element_type=jnp.float32)
    varv = jnp.sum(w2 * jnp.dot(Cz, w2, preferred_element_type=jnp.float32),
                   0, keepdims=True)
    y = g2_ref[...] * (v_ref[...] - mv) * lax.rsqrt(varv + 1e-5) + bb2_ref[...]
    y = y + hl_ref[...]
    rid = pid * BN + lax.broadcasted_iota(jnp.int32, (BN, 16), 0)
    y = jnp.where(rid < N, y, 0.0)

    @pl.when(pid == 0)
    def _():
        acc[...] = jnp.zeros_like(acc)

    acc[...] += jnp.sum(y, 0, keepdims=True)

    @pl.when(pid == NB - 1)
    def _():
        o_ref[...] = acc[...]


def _phaseC_last(v, hl, z1, z2, w2p, b2p, g2p, bb2p):
    return pl.pallas_call(
        _phC_last_body,
        grid=(NB,),
        in_specs=[pl.BlockSpec((BN, 16), lambda i: (i, 0)),
                  pl.BlockSpec((BN, 16), lambda i: (i, 0)),
                  pl.BlockSpec((1, H), lambda i: (0, 0)),
                  pl.BlockSpec((H, H), lambda i: (0, 0)),
                  pl.BlockSpec((H, 16), lambda i: (0, 0)),
                  pl.BlockSpec((1, 16), lambda i: (0, 0)),
                  pl.BlockSpec((1, 16), lambda i: (0, 0)),
                  pl.BlockSpec((1, 16), lambda i: (0, 0))],
        out_specs=pl.BlockSpec((1, 16), lambda i: (0, 0)),
        out_shape=jax.ShapeDtypeStruct((1, 16), jnp.float32),
        scratch_shapes=[pltpu.VMEM((1, 16), jnp.float32)],
    )(v, hl, z1, z2, w2p, b2p, g2p, bb2p)


def _vn_body(pool_ref, vn_ref, w1_ref, b1_ref, g1_ref, bb1_ref,
             w2_ref, b2_ref, g2_ref, bb2_ref, o_ref):
    vt = pool_ref[0] + pool_ref[1] + vn_ref[...]
    uu = jnp.dot(vt, w1_ref[...], preferred_element_type=jnp.float32) + b1_ref[...]
    mu = jnp.mean(uu, 0, keepdims=True)
    var = jnp.mean(uu * uu, 0, keepdims=True) - mu * mu
    uu = jnp.maximum(g1_ref[...] * (uu - mu) * lax.rsqrt(var + 1e-5)
                     + bb1_ref[...], 0.0)
    u2 = jnp.dot(uu, w2_ref[...], preferred_element_type=jnp.float32) + b2_ref[...]
    mu2 = jnp.mean(u2, 0, keepdims=True)
    var2 = jnp.mean(u2 * u2, 0, keepdims=True) - mu2 * mu2
    u2 = jnp.maximum(g2_ref[...] * (u2 - mu2) * lax.rsqrt(var2 + 1e-5)
                     + bb2_ref[...], 0.0)
    o_ref[...] = vn_ref[...] + u2


def _vn_update(pool, vn, vw1p, vb1, vg1, vbb1, vw2p, vb2p, vg2p, vbb2p):
    return pl.pallas_call(
        _vn_body,
        out_shape=jax.ShapeDtypeStruct((B, 16), jnp.float32),
    )(pool, vn, vw1p, vb1, vg1, vbb1, vw2p, vb2p, vg2p, vbb2p)


def kernel(x, edge_index, edge_attr, batch, atom_emb, bond_emb, eps, gin_w1,
           gin_b1, gin_bn1_g, gin_bn1_b, gin_w2, gin_b2, out_bn_g, out_bn_b,
           vn_w1, vn_b1, vn_bn1_g, vn_bn1_b, vn_w2, vn_b2, vn_bn2_g, vn_bn2_b):
    src = edge_index[0]
    dst = edge_index[1]

    lut16 = jnp.tile(jnp.pad(
        (bond_emb[0][:5][:, None, None, :] + bond_emb[1][:5][None, :, None, :]
         + bond_emb[2][:5][None, None, :, :]).reshape(125, D),
        ((0, 0), (0, 7))), (LUT_REP, 1))
    eidx = (edge_attr[:, 0] * 25 + edge_attr[:, 1] * 5 + edge_attr[:, 2]
            + 125 * (jnp.arange(E, dtype=jnp.int32) % LUT_REP))
    x_p = jnp.pad(x, ((0, NP - N), (0, 0)))
    batch_p = jnp.pad(batch, (0, NP - N))
    zeros = jnp.zeros((NP, 16), jnp.float32)
    atom16 = jnp.pad(atom_emb, ((0, 0), (0, 0), (0, 7)))
    w1p = jnp.pad(gin_w1, ((0, 0), (0, 7), (0, 0)))
    b1 = gin_b1[:, None, :]
    g1 = gin_bn1_g[:, None, :]
    bb1 = gin_bn1_b[:, None, :]
    w2p = jnp.pad(gin_w2, ((0, 0), (0, 0), (0, 7)))
    b2p = jnp.pad(gin_b2, ((0, 0), (0, 7)))[:, None, :]
    g2p = jnp.pad(out_bn_g, ((0, 0), (0, 7)))[:, None, :]
    bb2p = jnp.pad(out_bn_b, ((0, 0), (0, 7)))[:, None, :]
    vw1p = jnp.pad(vn_w1, ((0, 0), (0, 7), (0, 0)))
    vb1 = vn_b1[:, None, :]
    vg1 = vn_bn1_g[:, None, :]
    vbb1 = vn_bn1_b[:, None, :]
    vw2p = jnp.pad(vn_w2, ((0, 0), (0, 0), (0, 7)))
    vb2p = jnp.pad(vn_b2, ((0, 0), (0, 7)))[:, None, :]
    vg2p = jnp.pad(vn_bn2_g, ((0, 0), (0, 7)))[:, None, :]
    vbb2p = jnp.pad(vn_bn2_b, ((0, 0), (0, 7)))[:, None, :]

    sei = jnp.concatenate([src.reshape(SUP, SE), eidx.reshape(SUP, SE)], 1)
    dst3 = dst.reshape(SUP, SK, EC)
    batch3 = batch_p.reshape(N_SUP, SK, EC)

    hl = _embed(x_p, atom16)
    vn = jnp.zeros((B, 16), jnp.float32)
    out = None
    for l in range(L):
        agg, pool = _message(hl, sei, dst3, lut16, batch3, zeros)
        eps_l = eps[l].reshape(1, 1)
        u, s1, s2 = _phaseA(hl, agg, w1p[l], b1[l], eps_l)
        v, z1, z2 = _phaseB(u, s1, s2, w1p[l], b1[l], g1[l], bb1[l],
                            w2p[l], b2p[l])
        if l < L - 1:
            vn = _vn_update(pool, vn, vw1p[l], vb1[l], vg1[l], vbb1[l],
                            vw2p[l], vb2p[l], vg2p[l], vbb2p[l])
            hl = _phaseC_mid(v, hl, z1, z2, w2p[l], b2p[l], g2p[l], bb2p[l],
                             vn, batch_p)
        else:
            out = _phaseC_last(v, hl, z1, z2, w2p[l], b2p[l], g2p[l], bb2p[l])
    return out[:, :D]


# speedup vs baseline: 33.5677x; 1.7225x over previous
import functools

import jax
import jax.numpy as jnp
from jax import lax
from jax.experimental import pallas as pl
from jax.experimental.pallas import tpu as pltpu
from jax.experimental.pallas import tpu_sc as plsc

N = 100000
E = 3200000
B = 512
D = 9
H = 18
L = 3

BN = 2048
NB = 49
NP = BN * NB
NC = 2
NS = 16
NW = NC * NS
EC = 128
SK = 2
SE = SK * EC
SUP = E // SE
SBODY = -(-(-(-SUP // NW)) // 2)
PEND0 = SUP % NW
assert SUP % (2 * NW) == PEND0 and 0 < PEND0 <= NW
N_SUP = NP // SE
N_ITERS = -(-N_SUP // NW)
ROWS_PER_TILE = NP // NS
LUT_REP = 32


def _sc_msg_body(hl_h, sei_h, batch_h, lut_h, zeros_h,
                 agg_o, pool_o,
                 idx_v, rows_v, msg_v,
                 agg_s, pool_s,
                 sem_g0, sem_g1, sem_s0, sem_s1, sem_i0, sem_i1):
    c = lax.axis_index("c")
    s = lax.axis_index("s")
    wid = s * NC + c
    sem_g = (sem_g0, sem_g1)
    sem_s = (sem_s0, sem_s1)
    sem_i = (sem_i0, sem_i1)

    pltpu.sync_copy(zeros_h.at[pl.ds(s * ROWS_PER_TILE, ROWS_PER_TILE)],
                    agg_s.at[pl.ds(s * ROWS_PER_TILE, ROWS_PER_TILE)])

    @pl.when(s == 0)
    def _():
        pltpu.sync_copy(zeros_h.at[pl.ds(0, B)], pool_s)

    plsc.subcore_barrier()

    def issue_idx(p, cid):
        pltpu.async_copy(sei_h.at[cid], idx_v.at[p], sem_i[p])

    def wait_idx(p):
        pltpu.make_async_copy(sei_h.at[0], idx_v.at[p], sem_i[p]).wait()

    def issue_gathers(p):
        for j in range(SK):
            pltpu.async_copy(hl_h.at[idx_v.at[p, j]],
                             rows_v.at[p, pl.ds(j * EC, EC)], sem_g[p])
            pltpu.async_copy(lut_h.at[idx_v.at[p, SK + j]],
                             msg_v.at[p, pl.ds(j * EC, EC)], sem_g[p])

    def drain_gathers(p):
        pltpu.make_async_copy(zeros_h.at[pl.ds(0, SE)], rows_v.at[p],
                              sem_g[p]).wait()
        pltpu.make_async_copy(zeros_h.at[pl.ds(0, SE)], msg_v.at[p],
                              sem_g[p]).wait()

    def issue_scatters(p):
        for j in range(SK):
            pltpu.async_copy(msg_v.at[p, pl.ds(j * EC, EC)],
                             agg_s.at[idx_v.at[p, 2 * SK + j]], sem_s[p],
                             add=True)

    def drain_scatters(p):
        pltpu.make_async_copy(zeros_h.at[pl.ds(0, SE)], msg_v.at[p],
                              sem_s[p]).wait()

    def compute(p):
        def crow(j, carry):
            r0 = j * 16
            for t in range(16):
                r = r0 + t
                msg_v[p, r] = jnp.maximum(rows_v[p, r] + msg_v[p, r], 0.0)
            return carry
        lax.fori_loop(0, SE // 16, crow, None)

    issue_idx(0, wid)
    wait_idx(0)
    issue_gathers(0)

    def body(k, carry):
        cid_a = wid + NW * 2 * k
        cid_b = cid_a + NW
        cid_n = cid_a + 2 * NW

        @pl.when(jnp.logical_and(k > 0, cid_a - NW < SUP))
        def _():
            drain_scatters(1)

        @pl.when(cid_a < SUP)
        def _():
            @pl.when(cid_b < SUP)
            def _():
                issue_idx(1, cid_b)

            drain_gathers(0)

            @pl.when(cid_b < SUP)
            def _():
                wait_idx(1)
                issue_gathers(1)

            compute(0)
            issue_scatters(0)

        @pl.when(cid_b < SUP)
        def _():
            drain_scatters(0)

            @pl.when(cid_n < SUP)
            def _():
                issue_idx(0, cid_n)

            drain_gathers(1)

            @pl.when(cid_n < SUP)
            def _():
                wait_idx(0)
                issue_gathers(0)

            compute(1)
            issue_scatters(1)
        return carry

    lax.fori_loop(0, SBODY, body, None)

    @pl.when(wid < PEND0)
    def _():
        drain_scatters(0)

    def node_step(i, carry):
        cid = wid + NW * i

        @pl.when(cid < N_SUP)
        def _():
            pltpu.sync_copy(hl_h.at[pl.ds(cid * SE, SE)], rows_v.at[0])
            pltpu.sync_copy(batch_h.at[cid], idx_v.at[0, pl.ds(0, SK)])
            for j in range(SK):
                pltpu.sync_copy(rows_v.at[0, pl.ds(j * EC, EC)],
                                pool_s.at[idx_v.at[0, j]], add=True)
        return carry

    lax.fori_loop(0, N_ITERS, node_step, None)

    plsc.subcore_barrier()
    pltpu.sync_copy(agg_s.at[pl.ds(s * ROWS_PER_TILE, ROWS_PER_TILE)],
                    agg_o.at[c, pl.ds(s * ROWS_PER_TILE, ROWS_PER_TILE)])

    @pl.when(s == 0)
    def _():
        pltpu.sync_copy(pool_s, pool_o.at[c])


def _message(hl, sei, batch3, lut16, zeros):
    mesh = plsc.VectorSubcoreMesh(core_axis_name="c", subcore_axis_name="s")
    fn = pl.kernel(
        _sc_msg_body, mesh=mesh,
        compiler_params=pltpu.CompilerParams(use_tc_tiling_on_sc=False),
        out_type=[jax.ShapeDtypeStruct((NC, NP, 16), jnp.float32),
                  jax.ShapeDtypeStruct((NC, B, 16), jnp.float32)],
        scratch_types=[
            pltpu.VMEM((2, 3 * SK, EC), jnp.int32),
            pltpu.VMEM((2, SE, 16), jnp.float32),
            pltpu.VMEM((2, SE, 16), jnp.float32),
            pltpu.VMEM_SHARED((NP, 16), jnp.float32),
            pltpu.VMEM_SHARED((B, 16), jnp.float32),
            pltpu.SemaphoreType.DMA,
            pltpu.SemaphoreType.DMA,
            pltpu.SemaphoreType.DMA,
            pltpu.SemaphoreType.DMA,
            pltpu.SemaphoreType.DMA,
            pltpu.SemaphoreType.DMA,
        ],
    )
    return fn(hl, sei, batch3, lut16, zeros)


def _embed_body(x_ref, tab_ref, o_ref):
    pid = pl.program_id(0)
    h = jnp.zeros((BN, 16), jnp.float32)
    for f in range(9):
        col = x_ref[:, f]
        oh = (col[:, None] ==
              lax.broadcasted_iota(jnp.int32, (BN, 120), 1)).astype(jnp.float32)
        h = h + jnp.dot(oh, tab_ref[f], preferred_element_type=jnp.float32)
    rid = pid * BN + lax.broadcasted_iota(jnp.int32, (BN, 16), 0)
    o_ref[...] = jnp.where(rid < N, h, 0.0)


def _embed(x_p, atom16):
    return pl.pallas_call(
        _embed_body,
        grid=(NB,),
        in_specs=[pl.BlockSpec((BN, 9), lambda i: (i, 0)),
                  pl.BlockSpec((9, 120, 16), lambda i: (0, 0, 0))],
        out_specs=pl.BlockSpec((BN, 16), lambda i: (i, 0)),
        out_shape=jax.ShapeDtypeStruct((NP, 16), jnp.float32),
    )(x_p, atom16)


def _phA_body(hl_ref, a0_ref, a1_ref, w1_ref, b1_ref, eps_ref,
              u_ref, s1_ref, s2_ref, s1a, s2a):
    pid = pl.program_id(0)
    t16 = (1.0 + eps_ref[0, 0]) * hl_ref[...] + a0_ref[0] + a1_ref[0]
    u_ref[...] = jnp.dot(t16, w1_ref[...],
                         preferred_element_type=jnp.float32) + b1_ref[...]
    ps1 = jnp.sum(t16, 0, keepdims=True)
    ps2 = lax.dot_general(t16, t16, (((0,), (0,)), ((), ())),
                          preferred_element_type=jnp.float32)

    @pl.when(pid == 0)
    def _():
        s1a[...] = jnp.zeros_like(s1a)
        s2a[...] = jnp.zeros_like(s2a)

    s1a[...] += ps1
    s2a[...] += ps2

    @pl.when(pid == NB - 1)
    def _():
        s1_ref[...] = s1a[...]
        s2_ref[...] = s2a[...]


def _phaseA(hl, agg, w1p, b1, eps_l):
    return pl.pallas_call(
        _phA_body,
        grid=(NB,),
        in_specs=[pl.BlockSpec((BN, 16), lambda i: (i, 0)),
                  pl.BlockSpec((1, BN, 16), lambda i: (0, i, 0)),
                  pl.BlockSpec((1, BN, 16), lambda i: (1, i, 0)),
                  pl.BlockSpec((16, H), lambda i: (0, 0)),
                  pl.BlockSpec((1, H), lambda i: (0, 0)),
                  pl.BlockSpec((1, 1), lambda i: (0, 0))],
        out_specs=[pl.BlockSpec((BN, H), lambda i: (i, 0)),
                   pl.BlockSpec((1, 16), lambda i: (0, 0)),
                   pl.BlockSpec((16, 16), lambda i: (0, 0))],
        out_shape=[jax.ShapeDtypeStruct((NP, H), jnp.float32),
                   jax.ShapeDtypeStruct((1, 16), jnp.float32),
                   jax.ShapeDtypeStruct((16, 16), jnp.float32)],
        scratch_shapes=[pltpu.VMEM((1, 16), jnp.float32),
                        pltpu.VMEM((16, 16), jnp.float32)],
    )(hl, agg, agg, w1p, b1, eps_l)


def _phB_body(u_ref, s1_ref, s2_ref, w1_ref, b1_ref, g1_ref, bb1_ref,
              w2_ref, b2_ref, v_ref, z1_ref, z2_ref, z1a, z2a):
    pid = pl.program_id(0)
    m16 = s1_ref[...] / N
    w1 = w1_ref[...]
    mu = jnp.dot(m16, w1, preferred_element_type=jnp.float32) + b1_ref[...]
    C = s2_ref[...] / N - lax.dot_general(m16, m16, (((0,), (0,)), ((), ())),
                                          preferred_element_type=jnp.float32)
    varu = jnp.sum(w1 * jnp.dot(C, w1, preferred_element_type=jnp.float32),
                   0, keepdims=True)
    z = jnp.maximum(g1_ref[...] * (u_ref[...] - mu) * lax.rsqrt(varu + 1e-5)
                    + bb1_ref[...], 0.0)
    rid = pid * BN + lax.broadcasted_iota(jnp.int32, (BN, H), 0)
    z = jnp.where(rid < N, z, 0.0)
    v_ref[...] = jnp.dot(z, w2_ref[...],
                         preferred_element_type=jnp.float32) + b2_ref[...]
    pz1 = jnp.sum(z, 0, keepdims=True)
    pz2 = lax.dot_general(z, z, (((0,), (0,)), ((), ())),
                          preferred_element_type=jnp.float32)

    @pl.when(pid == 0)
    def _():
        z1a[...] = jnp.zeros_like(z1a)
        z2a[...] = jnp.zeros_like(z2a)

    z1a[...] += pz1
    z2a[...] += pz2

    @pl.when(pid == NB - 1)
    def _():
        z1_ref[...] = z1a[...]
        z2_ref[...] = z2a[...]


def _phaseB(u, s1, s2, w1p, b1, g1, bb1, w2p, b2p):
    return pl.pallas_call(
        _phB_body,
        grid=(NB,),
        in_specs=[pl.BlockSpec((BN, H), lambda i: (i, 0)),
                  pl.BlockSpec((1, 16), lambda i: (0, 0)),
                  pl.BlockSpec((16, 16), lambda i: (0, 0)),
                  pl.BlockSpec((16, H), lambda i: (0, 0)),
                  pl.BlockSpec((1, H), lambda i: (0, 0)),
                  pl.BlockSpec((1, H), lambda i: (0, 0)),
                  pl.BlockSpec((1, H), lambda i: (0, 0)),
                  pl.BlockSpec((H, 16), lambda i: (0, 0)),
                  pl.BlockSpec((1, 16), lambda i: (0, 0))],
        out_specs=[pl.BlockSpec((BN, 16), lambda i: (i, 0)),
                   pl.BlockSpec((1, H), lambda i: (0, 0)),
                   pl.BlockSpec((H, H), lambda i: (0, 0))],
        out_shape=[jax.ShapeDtypeStruct((NP, 16), jnp.float32),
                   jax.ShapeDtypeStruct((1, H), jnp.float32),
                   jax.ShapeDtypeStruct((H, H), jnp.float32)],
        scratch_shapes=[pltpu.VMEM((1, H), jnp.float32),
                        pltpu.VMEM((H, H), jnp.float32)],
    )(u, s1, s2, w1p, b1, g1, bb1, w2p, b2p)


def _phC_mid_body(v_ref, hl_ref, z1_ref, z2_ref, w2_ref, b2_ref, g2_ref,
                  bb2_ref, pool_ref, vnin_ref, vw1_ref, vb1_ref, vg1_ref,
                  vbb1_ref, vw2_ref, vb2_ref, vg2_ref, vbb2_ref, batch_ref,
                  y_ref, vno_ref):
    pid = pl.program_id(0)
    vt = pool_ref[0] + pool_ref[1] + vnin_ref[...]
    uu = jnp.dot(vt, vw1_ref[...], preferred_element_type=jnp.float32) + vb1_ref[...]
    mu = jnp.mean(uu, 0, keepdims=True)
    var = jnp.mean(uu * uu, 0, keepdims=True) - mu * mu
    uu = jnp.maximum(vg1_ref[...] * (uu - mu) * lax.rsqrt(var + 1e-5)
                     + vbb1_ref[...], 0.0)
    u2 = jnp.dot(uu, vw2_ref[...], preferred_element_type=jnp.float32) + vb2_ref[...]
    mu2 = jnp.mean(u2, 0, keepdims=True)
    var2 = jnp.mean(u2 * u2, 0, keepdims=True) - mu2 * mu2
    u2 = jnp.maximum(vg2_ref[...] * (u2 - mu2) * lax.rsqrt(var2 + 1e-5)
                     + vbb2_ref[...], 0.0)
    vn_new = vnin_ref[...] + u2
    vno_ref[...] = vn_new
    mz = z1_ref[...] / N
    w2 = w2_ref[...]
    mv = jnp.dot(mz, w2, preferred_element_type=jnp.float32) + b2_ref[...]
    Cz = z2_ref[...] / N - lax.dot_general(mz, mz, (((0,), (0,)), ((), ())),
                                           preferred_element_type=jnp.float32)
    varv = jnp.sum(w2 * jnp.dot(Cz, w2, preferred_element_type=jnp.float32),
                   0, keepdims=True)
    y = g2_ref[...] * (v_ref[...] - mv) * lax.rsqrt(varv + 1e-5) + bb2_ref[...]
    y = jnp.maximum(y, 0.0)
    y = y + hl_ref[...]
    oh = (batch_ref[...][:, None] ==
          lax.broadcasted_iota(jnp.int32, (BN, B), 1)).astype(jnp.float32)
    y = y + jnp.dot(oh, vn_new, preferred_element_type=jnp.float32)
    rid = pid * BN + lax.broadcasted_iota(jnp.int32, (BN, 16), 0)
    y_ref[...] = jnp.where(rid < N, y, 0.0)


def _phaseC_mid(v, hl, z1, z2, w2p, b2p, g2p, bb2p, pool, vn, vw1p, vb1,
                vg1, vbb1, vw2p, vb2p, vg2p, vbb2p, batch_p):
    cst = lambda i: (0, 0)
    return pl.pallas_call(
        _phC_mid_body,
        grid=(NB,),
        in_specs=[pl.BlockSpec((BN, 16), lambda i: (i, 0)),
                  pl.BlockSpec((BN, 16), lambda i: (i, 0)),
                  pl.BlockSpec((1, H), cst),
                  pl.BlockSpec((H, H), cst),
                  pl.BlockSpec((H, 16), cst),
                  pl.BlockSpec((1, 16), cst),
                  pl.BlockSpec((1, 16), cst),
                  pl.BlockSpec((1, 16), cst),
                  pl.BlockSpec((2, B, 16), lambda i: (0, 0, 0)),
                  pl.BlockSpec((B, 16), cst),
                  pl.BlockSpec((16, H), cst),
                  pl.BlockSpec((1, H), cst),
                  pl.BlockSpec((1, H), cst),
                  pl.BlockSpec((1, H), cst),
                  pl.BlockSpec((H, 16), cst),
                  pl.BlockSpec((1, 16), cst),
                  pl.BlockSpec((1, 16), cst),
                  pl.BlockSpec((1, 16), cst),
                  pl.BlockSpec((BN,), lambda i: (i,))],
        out_specs=[pl.BlockSpec((BN, 16), lambda i: (i, 0)),
                   pl.BlockSpec((B, 16), cst)],
        out_shape=[jax.ShapeDtypeStruct((NP, 16), jnp.float32),
                   jax.ShapeDtypeStruct((B, 16), jnp.float32)],
    )(v, hl, z1, z2, w2p, b2p, g2p, bb2p, pool, vn, vw1p, vb1, vg1, vbb1,
      vw2p, vb2p, vg2p, vbb2p, batch_p)


def _phC_last_body(v_ref, hl_ref, z1_ref, z2_ref, w2_ref, b2_ref, g2_ref,
                   bb2_ref, o_ref, acc):
    pid = pl.program_id(0)
    mz = z1_ref[...] / N
    w2 = w2_ref[...]
    mv = jnp.dot(mz, w2, preferred_element_type=jnp.float32) + b2_ref[...]
    Cz = z2_ref[...] / N - lax.dot_general(mz, mz, (((0,), (0,)), ((), ())),
                                           preferred_element_type=jnp.float32)
    varv = jnp.sum(w2 * jnp.dot(Cz, w2, preferred_element_type=jnp.float32),
                   0, keepdims=True)
    y = g2_ref[...] * (v_ref[...] - mv) * lax.rsqrt(varv + 1e-5) + bb2_ref[...]
    y = y + hl_ref[...]
    rid = pid * BN + lax.broadcasted_iota(jnp.int32, (BN, 16), 0)
    y = jnp.where(rid < N, y, 0.0)

    @pl.when(pid == 0)
    def _():
        acc[...] = jnp.zeros_like(acc)

    acc[...] += jnp.sum(y, 0, keepdims=True)

    @pl.when(pid == NB - 1)
    def _():
        o_ref[...] = acc[...]


def _phaseC_last(v, hl, z1, z2, w2p, b2p, g2p, bb2p):
    return pl.pallas_call(
        _phC_last_body,
        grid=(NB,),
        in_specs=[pl.BlockSpec((BN, 16), lambda i: (i, 0)),
                  pl.BlockSpec((BN, 16), lambda i: (i, 0)),
                  pl.BlockSpec((1, H), lambda i: (0, 0)),
                  pl.BlockSpec((H, H), lambda i: (0, 0)),
                  pl.BlockSpec((H, 16), lambda i: (0, 0)),
                  pl.BlockSpec((1, 16), lambda i: (0, 0)),
                  pl.BlockSpec((1, 16), lambda i: (0, 0)),
                  pl.BlockSpec((1, 16), lambda i: (0, 0))],
        out_specs=pl.BlockSpec((1, 16), lambda i: (0, 0)),
        out_shape=jax.ShapeDtypeStruct((1, 16), jnp.float32),
        scratch_shapes=[pltpu.VMEM((1, 16), jnp.float32)],
    )(v, hl, z1, z2, w2p, b2p, g2p, bb2p)


def _vn_body(pool_ref, vn_ref, w1_ref, b1_ref, g1_ref, bb1_ref,
             w2_ref, b2_ref, g2_ref, bb2_ref, o_ref):
    vt = pool_ref[0] + pool_ref[1] + vn_ref[...]
    uu = jnp.dot(vt, w1_ref[...], preferred_element_type=jnp.float32) + b1_ref[...]
    mu = jnp.mean(uu, 0, keepdims=True)
    var = jnp.mean(uu * uu, 0, keepdims=True) - mu * mu
    uu = jnp.maximum(g1_ref[...] * (uu - mu) * lax.rsqrt(var + 1e-5)
                     + bb1_ref[...], 0.0)
    u2 = jnp.dot(uu, w2_ref[...], preferred_element_type=jnp.float32) + b2_ref[...]
    mu2 = jnp.mean(u2, 0, keepdims=True)
    var2 = jnp.mean(u2 * u2, 0, keepdims=True) - mu2 * mu2
    u2 = jnp.maximum(g2_ref[...] * (u2 - mu2) * lax.rsqrt(var2 + 1e-5)
                     + bb2_ref[...], 0.0)
    o_ref[...] = vn_ref[...] + u2


def _vn_update(pool, vn, vw1p, vb1, vg1, vbb1, vw2p, vb2p, vg2p, vbb2p):
    return pl.pallas_call(
        _vn_body,
        out_shape=jax.ShapeDtypeStruct((B, 16), jnp.float32),
    )(pool, vn, vw1p, vb1, vg1, vbb1, vw2p, vb2p, vg2p, vbb2p)


def kernel(x, edge_index, edge_attr, batch, atom_emb, bond_emb, eps, gin_w1,
           gin_b1, gin_bn1_g, gin_bn1_b, gin_w2, gin_b2, out_bn_g, out_bn_b,
           vn_w1, vn_b1, vn_bn1_g, vn_bn1_b, vn_w2, vn_b2, vn_bn2_g, vn_bn2_b):
    src = edge_index[0]
    dst = edge_index[1]

    lut16 = jnp.tile(jnp.pad(
        (bond_emb[0][:5][:, None, None, :] + bond_emb[1][:5][None, :, None, :]
         + bond_emb[2][:5][None, None, :, :]).reshape(125, D),
        ((0, 0), (0, 7))), (LUT_REP, 1))
    eidx = (edge_attr[:, 0] * 25 + edge_attr[:, 1] * 5 + edge_attr[:, 2]
            + 125 * (jnp.arange(E, dtype=jnp.int32) % LUT_REP))
    x_p = jnp.pad(x, ((0, NP - N), (0, 0)))
    batch_p = jnp.pad(batch, (0, NP - N))
    zeros = jnp.zeros((NP, 16), jnp.float32)
    atom16 = jnp.pad(atom_emb, ((0, 0), (0, 0), (0, 7)))
    w1p = jnp.pad(gin_w1, ((0, 0), (0, 7), (0, 0)))
    b1 = gin_b1[:, None, :]
    g1 = gin_bn1_g[:, None, :]
    bb1 = gin_bn1_b[:, None, :]
    w2p = jnp.pad(gin_w2, ((0, 0), (0, 0), (0, 7)))
    b2p = jnp.pad(gin_b2, ((0, 0), (0, 7)))[:, None, :]
    g2p = jnp.pad(out_bn_g, ((0, 0), (0, 7)))[:, None, :]
    bb2p = jnp.pad(out_bn_b, ((0, 0), (0, 7)))[:, None, :]
    vw1p = jnp.pad(vn_w1, ((0, 0), (0, 7), (0, 0)))
    vb1 = vn_b1[:, None, :]
    vg1 = vn_bn1_g[:, None, :]
    vbb1 = vn_bn1_b[:, None, :]
    vw2p = jnp.pad(vn_w2, ((0, 0), (0, 0), (0, 7)))
    vb2p = jnp.pad(vn_b2, ((0, 0), (0, 7)))[:, None, :]
    vg2p = jnp.pad(vn_bn2_g, ((0, 0), (0, 7)))[:, None, :]
    vbb2p = jnp.pad(vn_bn2_b, ((0, 0), (0, 7)))[:, None, :]

    sei = jnp.concatenate([src.reshape(SUP, SK, EC),
                           eidx.reshape(SUP, SK, EC),
                           dst.reshape(SUP, SK, EC)], 1)
    batch3 = batch_p.reshape(N_SUP, SK, EC)

    hl = _embed(x_p, atom16)
    vn = jnp.zeros((B, 16), jnp.float32)
    out = None
    for l in range(L):
        agg, pool = _message(hl, sei, batch3, lut16, zeros)
        eps_l = eps[l].reshape(1, 1)
        u, s1, s2 = _phaseA(hl, agg, w1p[l], b1[l], eps_l)
        v, z1, z2 = _phaseB(u, s1, s2, w1p[l], b1[l], g1[l], bb1[l],
                            w2p[l], b2p[l])
        if l < L - 1:
            hl, vn = _phaseC_mid(v, hl, z1, z2, w2p[l], b2p[l], g2p[l],
                                 bb2p[l], pool, vn, vw1p[l], vb1[l], vg1[l],
                                 vbb1[l], vw2p[l], vb2p[l], vg2p[l],
                                 vbb2p[l], batch_p)
        else:
            out = _phaseC_last(v, hl, z1, z2, w2p[l], b2p[l], g2p[l], bb2p[l])
    return out[:, :D]
